# baseline (speedup 1.0000x reference)
import functools

import jax
import jax.numpy as jnp
from jax import lax
from jax.experimental import pallas as pl
from jax.experimental.pallas import tpu as pltpu
from jax.experimental.pallas import tpu_sc as plsc

EMB = 128
BATCH = 16384
NC = 2
NS = 16
NW = NC * NS
B_PER_W = BATCH // NW
CHUNK = 256
NCHUNK = B_PER_W // CHUNK


POS_SRC = 8
POS_ROWS = 256
LANES = 16


def _sc_kernel(inp_hbm, pidx_hbm, win_hbm, wpos_hbm, out_hbm,
               idx_v, pidx_v, possrc_v, emb_v, pos_v, sem_g, sem_p, sem_w):
    wid = lax.axis_index("s") * NC + lax.axis_index("c")
    base = wid * B_PER_W
    pltpu.sync_copy(pidx_hbm.at[pl.ds(0, POS_SRC)], pidx_v)
    pos_cp = pltpu.async_copy(wpos_hbm.at[pidx_v], possrc_v, sem_p)
    pltpu.sync_copy(inp_hbm.at[pl.ds(base, B_PER_W)], idx_v)
    gathers = [
        pltpu.async_copy(win_hbm.at[idx_v.at[pl.ds(c * CHUNK, CHUNK)]],
                         emb_v.at[pl.ds(c * CHUNK, CHUNK)], sem_g)
        for c in range(NCHUNK)
    ]
    pos_cp.wait()

    def rep_body(r, regs):
        for j in range(EMB // LANES):
            pos_v[r, pl.ds(j * LANES, LANES)] = regs[j]
        return regs

    regs0 = tuple(possrc_v[0, pl.ds(j * LANES, LANES)]
                  for j in range(EMB // LANES))
    lax.fori_loop(0, POS_ROWS, rep_body, regs0, unroll=4)

    writes = [
        pltpu.async_copy(
            pos_v,
            out_hbm.at[pl.ds(base + k * POS_ROWS, POS_ROWS),
                       pl.ds(EMB, EMB)],
            sem_w)
        for k in range(B_PER_W // POS_ROWS)
    ]
    for g in gathers:
        g.wait()
    writes.append(
        pltpu.async_copy(emb_v,
                         out_hbm.at[pl.ds(base, B_PER_W), pl.ds(0, EMB)],
                         sem_w))
    for w in writes:
        w.wait()


def kernel(input, input_position, W_in, W_pos):
    inp = input.astype(jnp.int32)
    pos_idx = jnp.full((POS_SRC,), 0, jnp.int32) + jnp.asarray(
        input_position, jnp.int32)

    mesh = plsc.VectorSubcoreMesh(core_axis_name="c", subcore_axis_name="s")
    run = functools.partial(
        pl.kernel,
        mesh=mesh,
        out_type=jax.ShapeDtypeStruct((BATCH, 2 * EMB), jnp.float32),
        scratch_types=[
            pltpu.VMEM((B_PER_W,), jnp.int32),
            pltpu.VMEM((POS_SRC,), jnp.int32),
            pltpu.VMEM((POS_SRC, EMB), jnp.float32),
            pltpu.VMEM((B_PER_W, EMB), jnp.float32),
            pltpu.VMEM((POS_ROWS, EMB), jnp.float32),
            pltpu.SemaphoreType.DMA,
            pltpu.SemaphoreType.DMA,
            pltpu.SemaphoreType.DMA,
        ],
    )(_sc_kernel)
    return run(inp, pos_idx, W_in, W_pos)


# speedup vs baseline: 1.1386x; 1.1386x over previous
import functools

import jax
import jax.numpy as jnp
from jax import lax
from jax.experimental import pallas as pl
from jax.experimental.pallas import tpu as pltpu
from jax.experimental.pallas import tpu_sc as plsc

EMB = 128
BATCH = 16384
NC = 2
NS = 16
NW = NC * NS
B_PER_W = BATCH // NW
CHUNK = 128
NCHUNK = B_PER_W // CHUNK


POS_SRC = 8
POS_ROWS = 256
LANES = 16


def _sc_kernel(inp_hbm, pidx_hbm, win_hbm, wpos_hbm, out_hbm,
               idx_v, pidx_v, possrc_v, emb_v, pos_v, sem_g, sem_p, sem_w):
    wid = lax.axis_index("s") * NC + lax.axis_index("c")
    base = wid * B_PER_W
    pltpu.sync_copy(inp_hbm.at[pl.ds(base, B_PER_W)], idx_v)
    pltpu.sync_copy(pidx_hbm.at[pl.ds(0, POS_SRC)], pidx_v)
    gathers = [
        pltpu.async_copy(win_hbm.at[idx_v.at[pl.ds(c * CHUNK, CHUNK)]],
                         emb_v.at[pl.ds(c * CHUNK, CHUNK)], sem_g)
        for c in range(NCHUNK)
    ]
    pltpu.async_copy(wpos_hbm.at[pidx_v], possrc_v, sem_p).wait()

    def rep_body(r, regs):
        for j in range(EMB // LANES):
            pos_v[r, pl.ds(j * LANES, LANES)] = regs[j]
        return regs

    regs0 = tuple(possrc_v[0, pl.ds(j * LANES, LANES)]
                  for j in range(EMB // LANES))
    lax.fori_loop(0, POS_ROWS, rep_body, regs0, unroll=4)

    for g in gathers:
        g.wait()
    writes = [
        pltpu.async_copy(emb_v,
                         out_hbm.at[pl.ds(base, B_PER_W), pl.ds(0, EMB)],
                         sem_w)
    ]
    for k in range(B_PER_W // POS_ROWS):
        writes.append(
            pltpu.async_copy(
                pos_v,
                out_hbm.at[pl.ds(base + k * POS_ROWS, POS_ROWS),
                           pl.ds(EMB, EMB)],
                sem_w))
    for w in writes:
        w.wait()


def kernel(input, input_position, W_in, W_pos):
    inp = input.astype(jnp.int32)
    pos_idx = jnp.full((POS_SRC,), 0, jnp.int32) + jnp.asarray(
        input_position, jnp.int32)

    mesh = plsc.VectorSubcoreMesh(core_axis_name="c", subcore_axis_name="s")
    run = functools.partial(
        pl.kernel,
        mesh=mesh,
        out_type=jax.ShapeDtypeStruct((BATCH, 2 * EMB), jnp.float32),
        scratch_types=[
            pltpu.VMEM((B_PER_W,), jnp.int32),
            pltpu.VMEM((POS_SRC,), jnp.int32),
            pltpu.VMEM((POS_SRC, EMB), jnp.float32),
            pltpu.VMEM((B_PER_W, EMB), jnp.float32),
            pltpu.VMEM((POS_ROWS, EMB), jnp.float32),
            pltpu.SemaphoreType.DMA,
            pltpu.SemaphoreType.DMA,
            pltpu.SemaphoreType.DMA,
        ],
    )(_sc_kernel)
    return run(inp, pos_idx, W_in, W_pos)


# speedup vs baseline: 1.1409x; 1.0020x over previous
import functools

import jax
import jax.numpy as jnp
from jax import lax
from jax.experimental import pallas as pl
from jax.experimental.pallas import tpu as pltpu
from jax.experimental.pallas import tpu_sc as plsc

EMB = 128
BATCH = 16384
NC = 2
NS = 16
NW = NC * NS
B_PER_W = BATCH // NW
CHUNK = 64
NCHUNK = B_PER_W // CHUNK


POS_SRC = 8
POS_ROWS = 256
LANES = 16


def _sc_kernel(inp_hbm, pidx_hbm, win_hbm, wpos_hbm, out_hbm,
               idx_v, pidx_v, possrc_v, emb_v, pos_v, sem_g, sem_p, sem_w):
    wid = lax.axis_index("s") * NC + lax.axis_index("c")
    base = wid * B_PER_W
    pltpu.sync_copy(inp_hbm.at[pl.ds(base, B_PER_W)], idx_v)
    pltpu.sync_copy(pidx_hbm.at[pl.ds(0, POS_SRC)], pidx_v)
    gathers = [
        pltpu.async_copy(win_hbm.at[idx_v.at[pl.ds(c * CHUNK, CHUNK)]],
                         emb_v.at[pl.ds(c * CHUNK, CHUNK)], sem_g)
        for c in range(NCHUNK)
    ]
    pltpu.async_copy(wpos_hbm.at[pidx_v], possrc_v, sem_p).wait()

    def rep_body(r, regs):
        for j in range(EMB // LANES):
            pos_v[r, pl.ds(j * LANES, LANES)] = regs[j]
        return regs

    regs0 = tuple(possrc_v[0, pl.ds(j * LANES, LANES)]
                  for j in range(EMB // LANES))
    lax.fori_loop(0, POS_ROWS, rep_body, regs0, unroll=4)

    for g in gathers:
        g.wait()
    writes = [
        pltpu.async_copy(emb_v,
                         out_hbm.at[pl.ds(base, B_PER_W), pl.ds(0, EMB)],
                         sem_w)
    ]
    for k in range(B_PER_W // POS_ROWS):
        writes.append(
            pltpu.async_copy(
                pos_v,
                out_hbm.at[pl.ds(base + k * POS_ROWS, POS_ROWS),
                           pl.ds(EMB, EMB)],
                sem_w))
    for w in writes:
        w.wait()


def kernel(input, input_position, W_in, W_pos):
    inp = input.astype(jnp.int32)
    pos_idx = jnp.full((POS_SRC,), 0, jnp.int32) + jnp.asarray(
        input_position, jnp.int32)

    mesh = plsc.VectorSubcoreMesh(core_axis_name="c", subcore_axis_name="s")
    run = functools.partial(
        pl.kernel,
        mesh=mesh,
        out_type=jax.ShapeDtypeStruct((BATCH, 2 * EMB), jnp.float32),
        scratch_types=[
            pltpu.VMEM((B_PER_W,), jnp.int32),
            pltpu.VMEM((POS_SRC,), jnp.int32),
            pltpu.VMEM((POS_SRC, EMB), jnp.float32),
            pltpu.VMEM((B_PER_W, EMB), jnp.float32),
            pltpu.VMEM((POS_ROWS, EMB), jnp.float32),
            pltpu.SemaphoreType.DMA,
            pltpu.SemaphoreType.DMA,
            pltpu.SemaphoreType.DMA,
        ],
    )(_sc_kernel)
    return run(inp, pos_idx, W_in, W_pos)


# speedup vs baseline: 1.1435x; 1.0023x over previous
import functools

import jax
import jax.numpy as jnp
from jax import lax
from jax.experimental import pallas as pl
from jax.experimental.pallas import tpu as pltpu
from jax.experimental.pallas import tpu_sc as plsc

EMB = 128
BATCH = 16384
NC = 2
NS = 16
NW = NC * NS
B_PER_W = BATCH // NW
CHUNK = 128
NCHUNK = B_PER_W // CHUNK


POS_SRC = 8
POS_ROWS = 256
LANES = 16


def _sc_kernel(inp_hbm, pidx_hbm, win_hbm, wpos_hbm, out_hbm,
               idx_v, pidx_v, possrc_v, emb_v, pos_v, sem_g, sem_p, sem_w):
    wid = lax.axis_index("s") * NC + lax.axis_index("c")
    base = wid * B_PER_W
    pltpu.sync_copy(inp_hbm.at[pl.ds(base, B_PER_W)], idx_v)
    pltpu.sync_copy(pidx_hbm.at[pl.ds(0, POS_SRC)], pidx_v)
    gathers = [
        pltpu.async_copy(win_hbm.at[idx_v.at[pl.ds(c * CHUNK, CHUNK)]],
                         emb_v.at[pl.ds(c * CHUNK, CHUNK)], sem_g)
        for c in range(NCHUNK)
    ]
    pltpu.async_copy(wpos_hbm.at[pidx_v], possrc_v, sem_p).wait()

    def rep_body(r, regs):
        for j in range(EMB // LANES):
            pos_v[r, pl.ds(j * LANES, LANES)] = regs[j]
        return regs

    regs0 = tuple(possrc_v[0, pl.ds(j * LANES, LANES)]
                  for j in range(EMB // LANES))
    lax.fori_loop(0, POS_ROWS, rep_body, regs0, unroll=4)

    for g in gathers:
        g.wait()
    writes = [
        pltpu.async_copy(emb_v,
                         out_hbm.at[pl.ds(base, B_PER_W), pl.ds(0, EMB)],
                         sem_w)
    ]
    for k in range(B_PER_W // POS_ROWS):
        writes.append(
            pltpu.async_copy(
                pos_v,
                out_hbm.at[pl.ds(base + k * POS_ROWS, POS_ROWS),
                           pl.ds(EMB, EMB)],
                sem_w))
    for w in writes:
        w.wait()


def kernel(input, input_position, W_in, W_pos):
    inp = input.astype(jnp.int32)
    pos_idx = jnp.full((POS_SRC,), 0, jnp.int32) + jnp.asarray(
        input_position, jnp.int32)

    mesh = plsc.VectorSubcoreMesh(core_axis_name="c", subcore_axis_name="s")
    run = functools.partial(
        pl.kernel,
        mesh=mesh,
        out_type=jax.ShapeDtypeStruct((BATCH, 2 * EMB), jnp.float32),
        scratch_types=[
            pltpu.VMEM((B_PER_W,), jnp.int32),
            pltpu.VMEM((POS_SRC,), jnp.int32),
            pltpu.VMEM((POS_SRC, EMB), jnp.float32),
            pltpu.VMEM((B_PER_W, EMB), jnp.float32),
            pltpu.VMEM((POS_ROWS, EMB), jnp.float32),
            pltpu.SemaphoreType.DMA,
            pltpu.SemaphoreType.DMA,
            pltpu.SemaphoreType.DMA,
        ],
    )(_sc_kernel)
    return run(inp, pos_idx, W_in, W_pos)
